# baseline (speedup 1.0000x reference)
import functools
import jax
import jax.numpy as jnp
from jax import lax
from jax.experimental import pallas as pl
from jax.experimental.pallas import tpu as pltpu
from jax.experimental.pallas import tpu_sc as plsc

HIDDEN = 4096
N_EXPERTS = 128
TOP_K = 8
N_TOK = 32768
BT = 1024
N_CHUNK = 8
CT = N_TOK // N_CHUNK
NC, NS, L = 2, 16, 16
NW = NC * NS
TPW = CT // NW


def _score_body(hs_ref, w_ref, bias_ref, s_ref):
    hs = hs_ref[...].astype(jnp.bfloat16)
    w = w_ref[...]
    logits_t = jax.lax.dot_general(
        w, hs, (((1,), (1,)), ((), ())),
        preferred_element_type=jnp.float32)
    s_ref[...] = jax.nn.sigmoid(logits_t) + bias_ref[...]


def _tc_scores_t(hs, w, bias, c):
    nb = CT // BT
    return pl.pallas_call(
        _score_body,
        grid=(nb,),
        in_specs=[
            pl.BlockSpec((BT, HIDDEN), lambda i: (c * nb + i, 0)),
            pl.BlockSpec((N_EXPERTS, HIDDEN), lambda i: (0, 0)),
            pl.BlockSpec((N_EXPERTS, 1), lambda i: (0, 0)),
        ],
        out_specs=pl.BlockSpec((N_EXPERTS, BT), lambda i: (0, i)),
        out_shape=jax.ShapeDtypeStruct((N_EXPERTS, CT), jnp.float32),
    )(hs, w, bias)


def _make_sc_topk():
    mesh = plsc.VectorSubcoreMesh(core_axis_name="c", subcore_axis_name="s")

    @functools.partial(
        pl.kernel,
        out_type=(jax.ShapeDtypeStruct((TOP_K, CT), jnp.int32),
                  jax.ShapeDtypeStruct((TOP_K, CT), jnp.float32)),
        mesh=mesh,
        scratch_types=[pltpu.VMEM((N_EXPERTS, TPW), jnp.float32),
                       pltpu.VMEM((TOP_K, TPW), jnp.int32),
                       pltpu.VMEM((TOP_K, TPW), jnp.float32)],
    )
    def sc_topk(st_hbm, idx_hbm, w_hbm, sv, iv, wv):
        wid = lax.axis_index("s") * NC + lax.axis_index("c")
        base = wid * TPW
        pltpu.sync_copy(st_hbm.at[:, pl.ds(base, TPW)], sv)

        def group(g, carry):
            off = g * L
            zf = jnp.zeros((L,), jnp.float32)
            zi = jnp.zeros((L,), jnp.int32)
            init = (zf,) * TOP_K + (zi,) * TOP_K

            def estep(e, st):
                s8 = list(st[:TOP_K])
                i8 = list(st[TOP_K:])
                v = sv[e, pl.ds(off, L)]
                vi = zi + e
                for j in range(TOP_K):
                    m = v > s8[j]
                    ns = jnp.where(m, v, s8[j])
                    ni = jnp.where(m, vi, i8[j])
                    v = jnp.where(m, s8[j], v)
                    vi = jnp.where(m, i8[j], vi)
                    s8[j] = ns
                    i8[j] = ni
                return tuple(s8) + tuple(i8)

            st = lax.fori_loop(0, N_EXPERTS, estep, init)
            s8 = st[:TOP_K]
            i8 = st[TOP_K:]
            tot = s8[0]
            for j in range(1, TOP_K):
                tot = tot + s8[j]
            den = tot + 1e-20
            for j in range(TOP_K):
                iv[j, pl.ds(off, L)] = i8[j]
                wv[j, pl.ds(off, L)] = s8[j] / den
            return carry

        lax.fori_loop(0, TPW // L, group, 0)
        pltpu.sync_copy(iv, idx_hbm.at[:, pl.ds(base, TPW)])
        pltpu.sync_copy(wv, w_hbm.at[:, pl.ds(base, TPW)])

    return sc_topk


def kernel(hidden_states, weight, e_score_correction_bias):
    hs = hidden_states.reshape(-1, HIDDEN).astype(jnp.float32)
    w = weight.astype(jnp.float32).astype(jnp.bfloat16)
    bias = e_score_correction_bias.reshape(N_EXPERTS, 1).astype(jnp.float32)

    sc_topk = _make_sc_topk()
    idx_parts = []
    w_parts = []
    for c in range(N_CHUNK):
        st = _tc_scores_t(hs, w, bias, c)
        ic, wc = sc_topk(st)
        idx_parts.append(ic)
        w_parts.append(wc)
    return (jnp.concatenate(idx_parts, axis=1).T,
            jnp.concatenate(w_parts, axis=1).T)


# speedup vs baseline: 1.0602x; 1.0602x over previous
import functools
import jax
import jax.numpy as jnp
from jax import lax
from jax.experimental import pallas as pl
from jax.experimental.pallas import tpu as pltpu
from jax.experimental.pallas import tpu_sc as plsc

HIDDEN = 4096
N_EXPERTS = 128
TOP_K = 8
N_TOK = 32768
BT = 1024
N_CHUNK = 2
CT = N_TOK // N_CHUNK
NC, NS, L = 2, 16, 16
NW = NC * NS
TPW = CT // NW


def _score_body(hs_ref, w_ref, bias_ref, s_ref):
    hs = hs_ref[...].astype(jnp.bfloat16)
    w = w_ref[...]
    logits_t = jax.lax.dot_general(
        w, hs, (((1,), (1,)), ((), ())),
        preferred_element_type=jnp.float32)
    s_ref[...] = jax.nn.sigmoid(logits_t) + bias_ref[...]


def _tc_scores_t(hs, w, bias, c):
    nb = CT // BT
    return pl.pallas_call(
        _score_body,
        grid=(nb,),
        in_specs=[
            pl.BlockSpec((BT, HIDDEN), lambda i: (c * nb + i, 0)),
            pl.BlockSpec((N_EXPERTS, HIDDEN), lambda i: (0, 0)),
            pl.BlockSpec((N_EXPERTS, 1), lambda i: (0, 0)),
        ],
        out_specs=pl.BlockSpec((N_EXPERTS, BT), lambda i: (0, i)),
        out_shape=jax.ShapeDtypeStruct((N_EXPERTS, CT), jnp.float32),
    )(hs, w, bias)


def _make_sc_topk():
    mesh = plsc.VectorSubcoreMesh(core_axis_name="c", subcore_axis_name="s")

    @functools.partial(
        pl.kernel,
        out_type=(jax.ShapeDtypeStruct((TOP_K, CT), jnp.int32),
                  jax.ShapeDtypeStruct((TOP_K, CT), jnp.float32)),
        mesh=mesh,
        scratch_types=[pltpu.VMEM((N_EXPERTS, TPW), jnp.float32),
                       pltpu.VMEM((TOP_K, TPW), jnp.int32),
                       pltpu.VMEM((TOP_K, TPW), jnp.float32)],
    )
    def sc_topk(st_hbm, idx_hbm, w_hbm, sv, iv, wv):
        wid = lax.axis_index("s") * NC + lax.axis_index("c")
        base = wid * TPW
        pltpu.sync_copy(st_hbm.at[:, pl.ds(base, TPW)], sv)

        def group(g, carry):
            off = g * L
            zf = jnp.zeros((L,), jnp.float32)
            zi = jnp.zeros((L,), jnp.int32)
            init = (zf,) * TOP_K + (zi,) * TOP_K

            def estep(e, st):
                s8 = list(st[:TOP_K])
                i8 = list(st[TOP_K:])
                v = sv[e, pl.ds(off, L)]
                vi = zi + e
                for j in range(TOP_K):
                    m = v > s8[j]
                    ns = jnp.where(m, v, s8[j])
                    ni = jnp.where(m, vi, i8[j])
                    v = jnp.where(m, s8[j], v)
                    vi = jnp.where(m, i8[j], vi)
                    s8[j] = ns
                    i8[j] = ni
                return tuple(s8) + tuple(i8)

            st = lax.fori_loop(0, N_EXPERTS, estep, init)
            s8 = st[:TOP_K]
            i8 = st[TOP_K:]
            tot = s8[0]
            for j in range(1, TOP_K):
                tot = tot + s8[j]
            den = tot + 1e-20
            for j in range(TOP_K):
                iv[j, pl.ds(off, L)] = i8[j]
                wv[j, pl.ds(off, L)] = s8[j] / den
            return carry

        lax.fori_loop(0, TPW // L, group, 0)
        pltpu.sync_copy(iv, idx_hbm.at[:, pl.ds(base, TPW)])
        pltpu.sync_copy(wv, w_hbm.at[:, pl.ds(base, TPW)])

    return sc_topk


def kernel(hidden_states, weight, e_score_correction_bias):
    hs = hidden_states.reshape(-1, HIDDEN).astype(jnp.float32)
    w = weight.astype(jnp.float32).astype(jnp.bfloat16)
    bias = e_score_correction_bias.reshape(N_EXPERTS, 1).astype(jnp.float32)

    sc_topk = _make_sc_topk()
    idx_parts = []
    w_parts = []
    for c in range(N_CHUNK):
        st = _tc_scores_t(hs, w, bias, c)
        ic, wc = sc_topk(st)
        idx_parts.append(ic)
        w_parts.append(wc)
    return (jnp.concatenate(idx_parts, axis=1).T,
            jnp.concatenate(w_parts, axis=1).T)


# speedup vs baseline: 1.0784x; 1.0171x over previous
import functools
import jax
import jax.numpy as jnp
from jax import lax
from jax.experimental import pallas as pl
from jax.experimental.pallas import tpu as pltpu
from jax.experimental.pallas import tpu_sc as plsc

HIDDEN = 4096
N_EXPERTS = 128
TOP_K = 8
N_TOK = 32768
BT = 1024
N_CHUNK = 4
CT = N_TOK // N_CHUNK
NC, NS, L = 2, 16, 16
NW = NC * NS
TPW = CT // NW


def _score_body(hs_ref, w_ref, bias_ref, s_ref):
    hs = hs_ref[...].astype(jnp.bfloat16)
    w = w_ref[...]
    logits_t = jax.lax.dot_general(
        w, hs, (((1,), (1,)), ((), ())),
        preferred_element_type=jnp.float32)
    s_ref[...] = jax.nn.sigmoid(logits_t) + bias_ref[...]


def _tc_scores_t(hs, w, bias, c):
    nb = CT // BT
    return pl.pallas_call(
        _score_body,
        grid=(nb,),
        in_specs=[
            pl.BlockSpec((BT, HIDDEN), lambda i: (c * nb + i, 0)),
            pl.BlockSpec((N_EXPERTS, HIDDEN), lambda i: (0, 0)),
            pl.BlockSpec((N_EXPERTS, 1), lambda i: (0, 0)),
        ],
        out_specs=pl.BlockSpec((N_EXPERTS, BT), lambda i: (0, i)),
        out_shape=jax.ShapeDtypeStruct((N_EXPERTS, CT), jnp.float32),
    )(hs, w, bias)


def _make_sc_topk():
    mesh = plsc.VectorSubcoreMesh(core_axis_name="c", subcore_axis_name="s")

    @functools.partial(
        pl.kernel,
        out_type=(jax.ShapeDtypeStruct((TOP_K, CT), jnp.int32),
                  jax.ShapeDtypeStruct((TOP_K, CT), jnp.float32)),
        mesh=mesh,
        scratch_types=[pltpu.VMEM((N_EXPERTS, TPW), jnp.float32),
                       pltpu.VMEM((TOP_K, TPW), jnp.int32),
                       pltpu.VMEM((TOP_K, TPW), jnp.float32)],
    )
    def sc_topk(st_hbm, idx_hbm, w_hbm, sv, iv, wv):
        wid = lax.axis_index("s") * NC + lax.axis_index("c")
        base = wid * TPW
        pltpu.sync_copy(st_hbm.at[:, pl.ds(base, TPW)], sv)

        def group(g, carry):
            off = g * L
            zf = jnp.zeros((L,), jnp.float32)
            zi = jnp.zeros((L,), jnp.int32)
            half = N_EXPERTS // 2

            def insert(v, vi, s8, i8):
                for j in range(TOP_K):
                    m = v > s8[j]
                    ns = jnp.where(m, v, s8[j])
                    ni = jnp.where(m, vi, i8[j])
                    v = jnp.where(m, s8[j], v)
                    vi = jnp.where(m, i8[j], vi)
                    s8[j] = ns
                    i8[j] = ni
                return s8, i8

            init = (zf,) * (2 * TOP_K) + (zi,) * (2 * TOP_K)

            def estep(e, st):
                sa = list(st[:TOP_K])
                sb = list(st[TOP_K:2 * TOP_K])
                ia = list(st[2 * TOP_K:3 * TOP_K])
                ib = list(st[3 * TOP_K:])
                va = sv[e, pl.ds(off, L)]
                vb = sv[e + half, pl.ds(off, L)]
                sa, ia = insert(va, zi + e, sa, ia)
                sb, ib = insert(vb, zi + (e + half), sb, ib)
                return tuple(sa) + tuple(sb) + tuple(ia) + tuple(ib)

            st = lax.fori_loop(0, half, estep, init)
            s8 = list(st[:TOP_K])
            i8 = list(st[2 * TOP_K:3 * TOP_K])
            sb = st[TOP_K:2 * TOP_K]
            ib = st[3 * TOP_K:]
            for j in range(TOP_K):
                s8, i8 = insert(sb[j], ib[j], s8, i8)
            tot = s8[0]
            for j in range(1, TOP_K):
                tot = tot + s8[j]
            den = tot + 1e-20
            for j in range(TOP_K):
                iv[j, pl.ds(off, L)] = i8[j]
                wv[j, pl.ds(off, L)] = s8[j] / den
            return carry

        lax.fori_loop(0, TPW // L, group, 0)
        pltpu.sync_copy(iv, idx_hbm.at[:, pl.ds(base, TPW)])
        pltpu.sync_copy(wv, w_hbm.at[:, pl.ds(base, TPW)])

    return sc_topk


def kernel(hidden_states, weight, e_score_correction_bias):
    hs = hidden_states.reshape(-1, HIDDEN).astype(jnp.float32)
    w = weight.astype(jnp.float32).astype(jnp.bfloat16)
    bias = e_score_correction_bias.reshape(N_EXPERTS, 1).astype(jnp.float32)

    sc_topk = _make_sc_topk()
    idx_parts = []
    w_parts = []
    for c in range(N_CHUNK):
        st = _tc_scores_t(hs, w, bias, c)
        ic, wc = sc_topk(st)
        idx_parts.append(ic)
        w_parts.append(wc)
    return (jnp.concatenate(idx_parts, axis=1).T,
            jnp.concatenate(w_parts, axis=1).T)


# speedup vs baseline: 1.0829x; 1.0042x over previous
import functools
import jax
import jax.numpy as jnp
from jax import lax
from jax.experimental import pallas as pl
from jax.experimental.pallas import tpu as pltpu
from jax.experimental.pallas import tpu_sc as plsc

HIDDEN = 4096
N_EXPERTS = 128
TOP_K = 8
N_TOK = 32768
BT = 1024
N_CHUNK = 4
CT = N_TOK // N_CHUNK
NC, NS, L = 2, 16, 16
NW = NC * NS
TPW = CT // NW


def _score_body(hs_ref, w_ref, bias_ref, s_ref):
    hs = hs_ref[...].astype(jnp.bfloat16)
    w = w_ref[...]
    logits_t = jax.lax.dot_general(
        w, hs, (((1,), (1,)), ((), ())),
        preferred_element_type=jnp.float32)
    s_ref[...] = jax.nn.sigmoid(logits_t) + bias_ref[...]


def _tc_scores_t(hs, w, bias, c):
    nb = CT // BT
    return pl.pallas_call(
        _score_body,
        grid=(nb,),
        in_specs=[
            pl.BlockSpec((BT, HIDDEN), lambda i: (c * nb + i, 0)),
            pl.BlockSpec((N_EXPERTS, HIDDEN), lambda i: (0, 0)),
            pl.BlockSpec((N_EXPERTS, 1), lambda i: (0, 0)),
        ],
        out_specs=pl.BlockSpec((N_EXPERTS, BT), lambda i: (0, i)),
        out_shape=jax.ShapeDtypeStruct((N_EXPERTS, CT), jnp.float32),
    )(hs, w, bias)


def _make_sc_topk():
    mesh = plsc.VectorSubcoreMesh(core_axis_name="c", subcore_axis_name="s")

    @functools.partial(
        pl.kernel,
        out_type=(jax.ShapeDtypeStruct((TOP_K, CT), jnp.int32),
                  jax.ShapeDtypeStruct((TOP_K, CT), jnp.float32)),
        mesh=mesh,
        scratch_types=[pltpu.VMEM((N_EXPERTS, TPW), jnp.float32),
                       pltpu.VMEM((TOP_K, TPW), jnp.int32),
                       pltpu.VMEM((TOP_K, TPW), jnp.float32)],
    )
    def sc_topk(st_hbm, idx_hbm, w_hbm, sv, iv, wv):
        wid = lax.axis_index("s") * NC + lax.axis_index("c")
        base = wid * TPW
        pltpu.sync_copy(st_hbm.at[:, pl.ds(base, TPW)], sv)

        def group(g, carry):
            off = g * L
            zf = jnp.zeros((L,), jnp.float32)
            zi = jnp.zeros((L,), jnp.int32)
            init = (zf,) * TOP_K + (zi,) * TOP_K

            def estep(e, st):
                s8 = list(st[:TOP_K])
                i8 = list(st[TOP_K:])
                v = sv[e, pl.ds(off, L)]
                vi = zi + e
                for j in range(TOP_K):
                    m = v > s8[j]
                    ns = jnp.where(m, v, s8[j])
                    ni = jnp.where(m, vi, i8[j])
                    v = jnp.where(m, s8[j], v)
                    vi = jnp.where(m, i8[j], vi)
                    s8[j] = ns
                    i8[j] = ni
                return tuple(s8) + tuple(i8)

            st = lax.fori_loop(0, N_EXPERTS, estep, init)
            s8 = st[:TOP_K]
            i8 = st[TOP_K:]
            tot = s8[0]
            for j in range(1, TOP_K):
                tot = tot + s8[j]
            den = tot + 1e-20
            for j in range(TOP_K):
                iv[j, pl.ds(off, L)] = i8[j]
                wv[j, pl.ds(off, L)] = s8[j] / den
            return carry

        lax.fori_loop(0, TPW // L, group, 0)
        pltpu.sync_copy(iv, idx_hbm.at[:, pl.ds(base, TPW)])
        pltpu.sync_copy(wv, w_hbm.at[:, pl.ds(base, TPW)])

    return sc_topk


def kernel(hidden_states, weight, e_score_correction_bias):
    hs = hidden_states.reshape(-1, HIDDEN).astype(jnp.float32)
    w = weight.astype(jnp.float32).astype(jnp.bfloat16)
    bias = e_score_correction_bias.reshape(N_EXPERTS, 1).astype(jnp.float32)

    sc_topk = _make_sc_topk()
    idx_parts = []
    w_parts = []
    for c in range(N_CHUNK):
        st = _tc_scores_t(hs, w, bias, c)
        ic, wc = sc_topk(st)
        idx_parts.append(ic)
        w_parts.append(wc)
    return (jnp.concatenate(idx_parts, axis=1).T,
            jnp.concatenate(w_parts, axis=1).T)
